# Initial kernel scaffold; baseline (speedup 1.0000x reference)
import jax
import jax.numpy as jnp
from jax.experimental import pallas as pl


def kernel(x, delta, theta, conv1_w, conv1_b, bn1_g, bn1_b, bn1_m, bn1_v, conv2_w, conv2_b, bn2_g, bn2_b, bn2_m, bn2_v, gat_w, gat_asrc, gat_adst, gat_bias, freq_w, freq_b, fc_w, fc_b):
    raise NotImplementedError("write your pallas kernel here")


# speedup vs baseline: 1.2509x; 1.2509x over previous
import functools

import jax
import jax.numpy as jnp
from jax.experimental import pallas as pl
from jax.experimental.pallas import tpu as pltpu

EPS = 1e-5
SLOPE = 0.2
N_NODES = 2048


def _shift(a, q, fill=0.0):
    if q == 0:
        return a
    r, _ = a.shape
    f = jnp.full((r, abs(q)), fill, a.dtype)
    if q > 0:
        return jnp.concatenate([a[:, q:], f], axis=1)
    return jnp.concatenate([f, a[:, :q]], axis=1)


def _fused_kernel(xp_ref, ft_ref, w1_ref, w2_ref, wg_ref, gbias_ref,
                  fw_ref, fb_ref, cw_ref, cb_ref, out_ref):
    ones = jnp.ones((1, N_NODES), jnp.float32)

    xph = [xp_ref[0, r] for r in range(4)]
    sh1 = {}
    for r in range(4):
        for q in (-1, 0, 1):
            sh1[(r, q)] = _shift(xph[r], q)
    h = []
    for p in range(4):
        rows = []
        for j in range(5):
            q, r = divmod(p + j - 2, 4)
            rows.append(sh1[(r, q)])
        rows.append(ones)
        a = jnp.concatenate(rows, axis=0)
        h.append(jax.nn.relu(
            jnp.dot(w1_ref[...], a, preferred_element_type=jnp.float32)))

    qsig = [jnp.maximum(h[0], h[1]), jnp.maximum(h[2], h[3])]

    sh2 = {}
    for r in range(2):
        for q in (-1, 0, 1):
            sh2[(r, q)] = _shift(qsig[r], q)
    g = []
    for r in range(2):
        rows = []
        for j in range(5):
            q, rr = divmod(r + j - 2, 2)
            rows.append(sh2[(rr, q)])
        rows.append(ones)
        b = jnp.concatenate(rows, axis=0)
        g.append(jax.nn.relu(
            jnp.dot(w2_ref[...], b, preferred_element_type=jnp.float32)))

    nodes = jnp.maximum(g[0], g[1])

    wxe = jnp.dot(wg_ref[...], nodes, preferred_element_type=jnp.float32)
    esrc = wxe[16:17, :]
    edst = wxe[17:18, :]
    logits = []
    for d in (-2, -1, 0, 1, 2):
        z = _shift(esrc, d, fill=-1e9) + edst
        logits.append(jnp.maximum(z, SLOPE * z))
    m = logits[0]
    for l in logits[1:]:
        m = jnp.maximum(m, l)
    exps = [jnp.exp(l - m) for l in logits]
    den = exps[0]
    for e in exps[1:]:
        den = den + e
    rden = 1.0 / den
    beta = jnp.zeros((1, N_NODES), jnp.float32)
    for k, d in enumerate((-2, -1, 0, 1, 2)):
        beta = beta + _shift(exps[k] * rden, -d)
    gat_row = jax.lax.dot_general(
        beta, wxe, (((1,), (1,)), ((), ())),
        preferred_element_type=jnp.float32)
    gmean = gat_row[:, :16] * (1.0 / N_NODES) + gbias_ref[...]

    freq = jax.nn.relu(
        jnp.dot(ft_ref[0], fw_ref[...], preferred_element_type=jnp.float32)
        + fb_ref[...])
    comb = jnp.concatenate([gmean, 1.5 * freq], axis=1)
    out_ref[0] = (
        jnp.dot(comb, cw_ref[...], preferred_element_type=jnp.float32)
        + cb_ref[...])


@jax.jit
def kernel(x, delta, theta, conv1_w, conv1_b, bn1_g, bn1_b, bn1_m, bn1_v,
           conv2_w, conv2_b, bn2_g, bn2_b, bn2_m, bn2_v,
           gat_w, gat_asrc, gat_adst, gat_bias, freq_w, freq_b, fc_w, fc_b):
    B, C_IN, T = x.shape

    xp = jnp.transpose(x.reshape(B, C_IN, T // 4, 4), (0, 3, 1, 2))

    s1 = bn1_g * jax.lax.rsqrt(bn1_v + EPS)
    t1 = (conv1_b - bn1_m) * s1 + bn1_b
    w1 = (conv1_w.transpose(0, 2, 1).reshape(32, 85)) * s1[:, None]
    w1 = jnp.concatenate([w1, t1[:, None]], axis=1)
    s2 = bn2_g * jax.lax.rsqrt(bn2_v + EPS)
    t2 = (conv2_b - bn2_m) * s2 + bn2_b
    w2 = (conv2_w.transpose(0, 2, 1).reshape(64, 160)) * s2[:, None]
    w2 = jnp.concatenate([w2, t2[:, None]], axis=1)

    wg = jnp.concatenate([gat_w.T,
                          (gat_w @ gat_asrc)[None, :],
                          (gat_w @ gat_adst)[None, :]], axis=0)

    ft = jnp.concatenate([delta, theta], axis=1)[:, None, :]

    grid = (B,)
    full = lambda s: pl.BlockSpec(s, lambda b: (0,) * len(s))
    out = pl.pallas_call(
        _fused_kernel,
        grid=grid,
        in_specs=[
            pl.BlockSpec((1, 4, C_IN, N_NODES), lambda b: (b, 0, 0, 0)),
            pl.BlockSpec((1, 1, 34), lambda b: (b, 0, 0)),
            full((32, 86)),
            full((64, 161)),
            full((18, 64)),
            full((1, 16)),
            full((34, 34)),
            full((1, 34)),
            full((50, 2)),
            full((1, 2)),
        ],
        out_specs=pl.BlockSpec((1, 1, 2), lambda b: (b, 0, 0)),
        out_shape=jax.ShapeDtypeStruct((B, 1, 2), jnp.float32),
        compiler_params=pltpu.CompilerParams(
            dimension_semantics=("parallel",),
            vmem_limit_bytes=100 * 1024 * 1024,
        ),
    )(xp, ft, w1, w2, wg, gat_bias[None, :], freq_w.T, freq_b[None, :],
      fc_w.T, fc_b[None, :])
    return out[:, 0, :]


# speedup vs baseline: 1.3818x; 1.1046x over previous
import functools

import jax
import jax.numpy as jnp
from jax.experimental import pallas as pl
from jax.experimental.pallas import tpu as pltpu

EPS = 1e-5
SLOPE = 0.2
N_NODES = 2048


def _shift(a, q, fill=0.0):
    if q == 0:
        return a
    r, _ = a.shape
    f = jnp.full((r, abs(q)), fill, a.dtype)
    if q > 0:
        return jnp.concatenate([a[:, q:], f], axis=1)
    return jnp.concatenate([f, a[:, :q]], axis=1)


def _fused_kernel(xp_ref, ft_ref, w1_ref, w2_ref, wg_ref, gbias_ref,
                  fw_ref, fb_ref, cw_ref, cb_ref, out_ref):
    ones = jnp.ones((1, N_NODES), jnp.bfloat16)

    xph = [xp_ref[0, r] for r in range(4)]
    sh1 = {}
    for r in range(4):
        for q in (-1, 0, 1):
            sh1[(r, q)] = _shift(xph[r], q)
    h = []
    for p in range(4):
        rows = []
        for j in range(5):
            q, r = divmod(p + j - 2, 4)
            rows.append(sh1[(r, q)])
        rows.append(ones)
        a = jnp.concatenate(rows, axis=0)
        h.append(jax.nn.relu(
            jnp.dot(w1_ref[...], a, preferred_element_type=jnp.float32)))

    qsig = [jnp.maximum(h[0], h[1]).astype(jnp.bfloat16),
            jnp.maximum(h[2], h[3]).astype(jnp.bfloat16)]

    sh2 = {}
    for r in range(2):
        for q in (-1, 0, 1):
            sh2[(r, q)] = _shift(qsig[r], q)
    g = []
    for r in range(2):
        rows = []
        for j in range(5):
            q, rr = divmod(r + j - 2, 2)
            rows.append(sh2[(rr, q)])
        rows.append(ones)
        b = jnp.concatenate(rows, axis=0)
        g.append(jax.nn.relu(
            jnp.dot(w2_ref[...], b, preferred_element_type=jnp.float32)))

    nodes = jnp.maximum(g[0], g[1]).astype(jnp.bfloat16)

    wxe = jnp.dot(wg_ref[...], nodes, preferred_element_type=jnp.float32)
    esrc = wxe[16:17, :]
    edst = wxe[17:18, :]
    logits = []
    for d in (-2, -1, 0, 1, 2):
        z = _shift(esrc, d, fill=-1e9) + edst
        logits.append(jnp.maximum(z, SLOPE * z))
    m = logits[0]
    for l in logits[1:]:
        m = jnp.maximum(m, l)
    exps = [jnp.exp(l - m) for l in logits]
    den = exps[0]
    for e in exps[1:]:
        den = den + e
    rden = 1.0 / den
    beta = jnp.zeros((1, N_NODES), jnp.float32)
    for k, d in enumerate((-2, -1, 0, 1, 2)):
        beta = beta + _shift(exps[k] * rden, -d)
    gat_row = jax.lax.dot_general(
        beta.astype(jnp.bfloat16), wxe.astype(jnp.bfloat16),
        (((1,), (1,)), ((), ())),
        preferred_element_type=jnp.float32)
    gmean = gat_row[:, :16] * (1.0 / N_NODES) + gbias_ref[...]

    freq = jax.nn.relu(
        jnp.dot(ft_ref[0], fw_ref[...], preferred_element_type=jnp.float32)
        + fb_ref[...])
    comb = jnp.concatenate([gmean, 1.5 * freq], axis=1)
    out_ref[0] = (
        jnp.dot(comb, cw_ref[...], preferred_element_type=jnp.float32)
        + cb_ref[...])


@jax.jit
def kernel(x, delta, theta, conv1_w, conv1_b, bn1_g, bn1_b, bn1_m, bn1_v,
           conv2_w, conv2_b, bn2_g, bn2_b, bn2_m, bn2_v,
           gat_w, gat_asrc, gat_adst, gat_bias, freq_w, freq_b, fc_w, fc_b):
    B, C_IN, T = x.shape

    xp = jnp.transpose(x.reshape(B, C_IN, T // 4, 4),
                       (0, 3, 1, 2)).astype(jnp.bfloat16)

    s1 = bn1_g * jax.lax.rsqrt(bn1_v + EPS)
    t1 = (conv1_b - bn1_m) * s1 + bn1_b
    w1 = (conv1_w.transpose(0, 2, 1).reshape(32, 85)) * s1[:, None]
    w1 = jnp.concatenate([w1, t1[:, None]], axis=1)
    s2 = bn2_g * jax.lax.rsqrt(bn2_v + EPS)
    t2 = (conv2_b - bn2_m) * s2 + bn2_b
    w2 = (conv2_w.transpose(0, 2, 1).reshape(64, 160)) * s2[:, None]
    w2 = jnp.concatenate([w2, t2[:, None]], axis=1)

    wg = jnp.concatenate([gat_w.T,
                          (gat_w @ gat_asrc)[None, :],
                          (gat_w @ gat_adst)[None, :]], axis=0)

    ft = jnp.concatenate([delta, theta], axis=1)[:, None, :]

    grid = (B,)
    full = lambda s: pl.BlockSpec(s, lambda b: (0,) * len(s))
    out = pl.pallas_call(
        _fused_kernel,
        grid=grid,
        in_specs=[
            pl.BlockSpec((1, 4, C_IN, N_NODES), lambda b: (b, 0, 0, 0)),
            pl.BlockSpec((1, 1, 34), lambda b: (b, 0, 0)),
            full((32, 86)),
            full((64, 161)),
            full((18, 64)),
            full((1, 16)),
            full((34, 34)),
            full((1, 34)),
            full((50, 2)),
            full((1, 2)),
        ],
        out_specs=pl.BlockSpec((1, 1, 2), lambda b: (b, 0, 0)),
        out_shape=jax.ShapeDtypeStruct((B, 1, 2), jnp.float32),
        compiler_params=pltpu.CompilerParams(
            dimension_semantics=("parallel",),
            vmem_limit_bytes=100 * 1024 * 1024,
        ),
    )(xp, ft, w1.astype(jnp.bfloat16), w2.astype(jnp.bfloat16),
      wg.astype(jnp.bfloat16), gat_bias[None, :], freq_w.T,
      freq_b[None, :], fc_w.T, fc_b[None, :])
    return out[:, 0, :]
